# Initial kernel scaffold; baseline (speedup 1.0000x reference)
import jax
import jax.numpy as jnp
from jax.experimental import pallas as pl


def kernel(features, edge_index, W_self0, W_neigh0, b0, W_self1, W_neigh1, b1, W_self2, W_neigh2, b2):
    raise NotImplementedError("write your pallas kernel here")


# speedup vs baseline: 6.2727x; 6.2727x over previous
import functools

import jax
import jax.numpy as jnp
from jax import lax
from jax.experimental import pallas as pl
from jax.experimental.pallas import tpu as pltpu
from jax.experimental.pallas import tpu_sc as plsc

_N = 10000
_D = 128
_E = 320000

_NC = 2
_NS = 16
_K = 128

_NPAD = 10240
_ROWS_PER_TILE = _NPAD // _NS
_DEG_PER_TILE = _NPAD // _NS
_ZROWS = 32


def _sc_agg_body(with_deg, u_hbm, src_hbm, dst_hbm, agg_out, deg_out,
                 idx_src, idx_dst, rows_v, ones_v, zbuf, dzero,
                 agg_sh, deg_sh, sem):
  c = lax.axis_index("c")
  s = lax.axis_index("s")

  zero16 = jnp.zeros((16,), jnp.float32)
  for r in range(_ZROWS):
    for l in range(_D // 16):
      zbuf[r, pl.ds(l * 16, 16)] = zero16
  if with_deg:
    for l in range(_DEG_PER_TILE // 16):
      dzero[pl.ds(l * 16, 16)] = zero16
    for l in range(_K // 16):
      ones_v[pl.ds(l * 16, 16)] = jnp.ones((16,), jnp.float32)

  row0 = s * _ROWS_PER_TILE
  for t in range(_ROWS_PER_TILE // _ZROWS):
    pltpu.sync_copy(zbuf, agg_sh.at[pl.ds(row0 + t * _ZROWS, _ZROWS)])
  if with_deg:
    pltpu.sync_copy(dzero, deg_sh.at[pl.ds(s * _DEG_PER_TILE, _DEG_PER_TILE)])

  plsc.subcore_barrier()

  chunks_per_core = _E // _NC // _K
  iters = (chunks_per_core + _NS - 1) // _NS

  def step(t, carry):
    j = s + t * _NS

    @pl.when(j < chunks_per_core)
    def _():
      base = c * (_E // _NC) + j * _K
      pltpu.sync_copy(src_hbm.at[pl.ds(base, _K)], idx_src)
      pltpu.sync_copy(dst_hbm.at[pl.ds(base, _K)], idx_dst)
      pltpu.async_copy(u_hbm.at[idx_src], rows_v, sem).wait()
      pltpu.sync_copy(rows_v, agg_sh.at[idx_dst], add=True)
      if with_deg:
        pltpu.sync_copy(ones_v, deg_sh.at[idx_dst], add=True)

    return carry

  lax.fori_loop(0, iters, step, 0)

  plsc.subcore_barrier()

  out_row0 = c * _NPAD + row0
  pltpu.sync_copy(agg_sh.at[pl.ds(row0, _ROWS_PER_TILE)],
                  agg_out.at[pl.ds(out_row0, _ROWS_PER_TILE)])
  if with_deg:
    d0 = s * _DEG_PER_TILE
    pltpu.sync_copy(deg_sh.at[pl.ds(d0, _DEG_PER_TILE)],
                    deg_out.at[pl.ds(c * _NPAD + d0, _DEG_PER_TILE)])


def _make_sc_agg(with_deg):
  mesh = plsc.VectorSubcoreMesh(core_axis_name="c", subcore_axis_name="s")
  if with_deg:
    out_type = (jax.ShapeDtypeStruct((_NC * _NPAD, _D), jnp.float32),
                jax.ShapeDtypeStruct((_NC * _NPAD,), jnp.float32))
  else:
    out_type = (jax.ShapeDtypeStruct((_NC * _NPAD, _D), jnp.float32),)
  scratch = [
      pltpu.VMEM((_K,), jnp.int32),
      pltpu.VMEM((_K,), jnp.int32),
      pltpu.VMEM((_K, _D), jnp.float32),
      pltpu.VMEM((_K,), jnp.float32),
      pltpu.VMEM((_ZROWS, _D), jnp.float32),
      pltpu.VMEM((_DEG_PER_TILE,), jnp.float32),
      pltpu.VMEM_SHARED((_NPAD, _D), jnp.float32),
      pltpu.VMEM_SHARED((_NPAD,), jnp.float32),
      pltpu.SemaphoreType.DMA,
  ]

  if with_deg:
    def body(u, src, dst, agg, deg, *scr):
      _sc_agg_body(True, u, src, dst, agg, deg, *scr)
  else:
    def body(u, src, dst, agg, *scr):
      _sc_agg_body(False, u, src, dst, agg, None, *scr)

  return pl.kernel(body, out_type=out_type, mesh=mesh, scratch_types=scratch)


_BLK = 1000


def _first_body(h_ref, wn_ref, ws_ref, b_ref, u_ref, s_ref):
  h = h_ref[...]
  u_ref[...] = jnp.dot(h, wn_ref[...], preferred_element_type=jnp.float32)
  s_ref[...] = jnp.dot(h, ws_ref[...],
                       preferred_element_type=jnp.float32) + b_ref[...]


def _mid_body(sp_ref, a0_ref, a1_ref, d0_ref, d1_ref, wn_ref, ws_ref, b_ref,
              u_ref, s_ref):
  inv = 1.0 / jnp.maximum(d0_ref[...] + d1_ref[...], 1.0)
  h = jnp.maximum(sp_ref[...] + (a0_ref[...] + a1_ref[...]) * inv, 0.0)
  u_ref[...] = jnp.dot(h, wn_ref[...], preferred_element_type=jnp.float32)
  s_ref[...] = jnp.dot(h, ws_ref[...],
                       preferred_element_type=jnp.float32) + b_ref[...]


def _last_body(sp_ref, a0_ref, a1_ref, d0_ref, d1_ref, o_ref):
  inv = 1.0 / jnp.maximum(d0_ref[...] + d1_ref[...], 1.0)
  o_ref[...] = jnp.maximum(
      sp_ref[...] + (a0_ref[...] + a1_ref[...]) * inv, 0.0)


_row_spec = pl.BlockSpec((_BLK, _D), lambda i: (i, 0))
_w_spec = pl.BlockSpec((_D, _D), lambda i: (0, 0))
_b_spec = pl.BlockSpec((1, _D), lambda i: (0, 0))
_deg_spec = pl.BlockSpec((_BLK, 1), lambda i: (i, 0))


def _tc_first(h, wn, ws, b):
  return pl.pallas_call(
      _first_body,
      grid=(_N // _BLK,),
      in_specs=[_row_spec, _w_spec, _w_spec, _b_spec],
      out_specs=[_row_spec, _row_spec],
      out_shape=[jax.ShapeDtypeStruct((_N, _D), jnp.float32)] * 2,
  )(h, wn, ws, b)


def _tc_mid(s_prev, a0, a1, d0, d1, wn, ws, b):
  return pl.pallas_call(
      _mid_body,
      grid=(_N // _BLK,),
      in_specs=[_row_spec, _row_spec, _row_spec, _deg_spec, _deg_spec,
                _w_spec, _w_spec, _b_spec],
      out_specs=[_row_spec, _row_spec],
      out_shape=[jax.ShapeDtypeStruct((_N, _D), jnp.float32)] * 2,
  )(s_prev, a0, a1, d0, d1, wn, ws, b)


def _tc_last(s_prev, a0, a1, d0, d1):
  return pl.pallas_call(
      _last_body,
      grid=(_N // _BLK,),
      in_specs=[_row_spec, _row_spec, _row_spec, _deg_spec, _deg_spec],
      out_specs=_row_spec,
      out_shape=jax.ShapeDtypeStruct((_N, _D), jnp.float32),
  )(s_prev, a0, a1, d0, d1)


@jax.jit
def _run(features, edge_index, wn, ws, bs):
  src = edge_index[0]
  dst = edge_index[1]
  sc_agg_deg = _make_sc_agg(True)
  sc_agg = _make_sc_agg(False)

  u, s = _tc_first(features, wn[0], ws[0], bs[0])
  agg, deg = sc_agg_deg(u, src, dst)
  d0 = deg[:_N, None]
  d1 = deg[_NPAD:_NPAD + _N, None]

  for i in (1, 2):
    u, s = _tc_mid(s, agg[:_N], agg[_NPAD:_NPAD + _N], d0, d1,
                   wn[i], ws[i], bs[i])
    (agg,) = sc_agg(u, src, dst)

  return _tc_last(s, agg[:_N], agg[_NPAD:_NPAD + _N], d0, d1)


def kernel(features, edge_index, W_self0, W_neigh0, b0, W_self1, W_neigh1,
           b1, W_self2, W_neigh2, b2):
  wn = (W_neigh0, W_neigh1, W_neigh2)
  ws = (W_self0, W_self1, W_self2)
  bs = (b0.reshape(1, _D), b1.reshape(1, _D), b2.reshape(1, _D))
  return _run(features, edge_index, wn, ws, bs)
